# Initial kernel scaffold; baseline (speedup 1.0000x reference)
import jax
import jax.numpy as jnp
from jax.experimental import pallas as pl


def kernel(x, table):
    raise NotImplementedError("write your pallas kernel here")


# speedup vs baseline: 4.5143x; 4.5143x over previous
import functools

import jax
import jax.numpy as jnp
from jax import lax
from jax.experimental import pallas as pl
from jax.experimental.pallas import tpu as pltpu
from jax.experimental.pallas import tpu_sc as plsc

EMBED_DIM = 64


@functools.partial(jax.jit, static_argnames=("batch",))
def _gather_rows(table, flat_idx, batch):
    info = plsc.get_sparse_core_info()
    nc, ns = info.num_cores, info.num_subcores
    nw = nc * ns
    b_per_w = batch // nw
    chunk = 640
    n_chunks = b_per_w // chunk
    mesh = plsc.VectorSubcoreMesh(core_axis_name="c", subcore_axis_name="s")

    @functools.partial(
        pl.kernel,
        out_type=jax.ShapeDtypeStruct((batch, EMBED_DIM), jnp.float32),
        mesh=mesh,
        scratch_types=[
            pltpu.VMEM((chunk,), jnp.int32),
            pltpu.VMEM((chunk, EMBED_DIM), jnp.float32),
            pltpu.SemaphoreType.DMA,
        ],
        compiler_params=pltpu.CompilerParams(use_tc_tiling_on_sc=False),
    )
    def body(table_hbm, idx_hbm, out_hbm, idx_v, rows_v, sem):
        wid = lax.axis_index("s") * nc + lax.axis_index("c")
        wbase = wid * b_per_w

        def chunk_body(i, carry):
            base = wbase + i * chunk
            pltpu.sync_copy(idx_hbm.at[pl.ds(base, chunk)], idx_v)
            pltpu.async_copy(table_hbm.at[idx_v], rows_v, sem).wait()
            pltpu.sync_copy(rows_v, out_hbm.at[pl.ds(base, chunk)])
            return carry

        lax.fori_loop(0, n_chunks, chunk_body, 0)

    return body(table, flat_idx)


def kernel(x, table):
    batch, hist = x.shape
    flat_idx = x.reshape(-1).astype(jnp.int32)
    out = _gather_rows(table, flat_idx, batch * hist)
    return out.reshape(batch, hist, EMBED_DIM)


# speedup vs baseline: 4.6007x; 1.0191x over previous
import functools

import jax
import jax.numpy as jnp
from jax import lax
from jax.experimental import pallas as pl
from jax.experimental.pallas import tpu as pltpu
from jax.experimental.pallas import tpu_sc as plsc

EMBED_DIM = 64
NBUF = 4
CHUNK = 400


@functools.partial(jax.jit, static_argnames=("batch",))
def _gather_rows(table, flat_idx, batch):
    info = plsc.get_sparse_core_info()
    nc, ns = info.num_cores, info.num_subcores
    nw = nc * ns
    b_per_w = batch // nw
    n_chunks = b_per_w // CHUNK
    n_groups = n_chunks // NBUF
    mesh = plsc.VectorSubcoreMesh(core_axis_name="c", subcore_axis_name="s")

    @functools.partial(
        pl.kernel,
        out_type=jax.ShapeDtypeStruct((batch, EMBED_DIM), jnp.float32),
        mesh=mesh,
        scratch_types=[
            pltpu.VMEM((b_per_w,), jnp.int32),
            pltpu.VMEM((NBUF, CHUNK, EMBED_DIM), jnp.float32),
            [pltpu.SemaphoreType.DMA] * NBUF,
            [pltpu.SemaphoreType.DMA] * NBUF,
        ],
        compiler_params=pltpu.CompilerParams(use_tc_tiling_on_sc=False),
    )
    def body(table_hbm, idx_hbm, out_hbm, idx_v, rows_v, gsems, ssems):
        wid = lax.axis_index("s") * nc + lax.axis_index("c")
        wbase = wid * b_per_w
        pltpu.sync_copy(idx_hbm.at[pl.ds(wbase, b_per_w)], idx_v)

        def gather_desc(i, b):
            return pltpu.make_async_copy(
                table_hbm.at[idx_v.at[pl.ds(i * CHUNK, CHUNK)]],
                rows_v.at[b],
                gsems[b],
            )

        def store_desc(i, b):
            return pltpu.make_async_copy(
                rows_v.at[b],
                out_hbm.at[pl.ds(wbase + i * CHUNK, CHUNK)],
                ssems[b],
            )

        for b in range(NBUF):
            gather_desc(b, b).start()

        def group(g, carry):
            for b in range(NBUF):
                i = g * NBUF + b
                gather_desc(i, b).wait()
                store_desc(i, b).start()
            for b in range(NBUF):
                nxt = (g + 1) * NBUF + b

                @pl.when(nxt < n_chunks)
                def _():
                    store_desc(g * NBUF + b, b).wait()
                    gather_desc(nxt, b).start()

            return carry

        lax.fori_loop(0, n_groups, group, 0)
        for b in range(NBUF):
            store_desc((n_groups - 1) * NBUF + b, b).wait()

    return body(table, flat_idx)


def kernel(x, table):
    batch, hist = x.shape
    flat_idx = x.reshape(-1).astype(jnp.int32)
    out = _gather_rows(table, flat_idx, batch * hist)
    return out.reshape(batch, hist, EMBED_DIM)
